# Initial kernel scaffold; baseline (speedup 1.0000x reference)
import jax
import jax.numpy as jnp
from jax.experimental import pallas as pl


def kernel(h_idx, r_idx, t_idx, entity_emb, relation_emb):
    raise NotImplementedError("write your pallas kernel here")


# speedup vs baseline: 2.1852x; 2.1852x over previous
import functools

import jax
import jax.numpy as jnp
from jax import lax
from jax.experimental import pallas as pl
from jax.experimental.pallas import tpu as pltpu
from jax.experimental.pallas import tpu_sc as plsc

_LANES = 16
_HALF = 64
_DIM = 128
_CHUNK = 128


def _trig_body(rel_ref, out_ref):
    p = rel_ref[...]
    out_ref[...] = jnp.concatenate([jnp.cos(p), jnp.sin(p)], axis=-1)


def _make_trig_table(relation_emb):
    n, hd = relation_emb.shape
    return pl.pallas_call(
        _trig_body,
        out_shape=jax.ShapeDtypeStruct((n, 2 * hd), jnp.float32),
    )(relation_emb)


def _sqrt_sc(x):
    i = lax.bitcast_convert_type(x, jnp.int32)
    i = 0x5F3759DF - (i >> 1)
    y = lax.bitcast_convert_type(i, jnp.float32)
    y = y * (1.5 - 0.5 * x * y * y)
    y = y * (1.5 - 0.5 * x * y * y)
    return x * y


def _sc_body(n_chunks, n_cores, h_idx, r_idx, t_idx, ent, cs, out,
             hidx_v, ridx_v, tidx_v, h_rows, t_rows, cs_rows, out_v, acc_scr,
             sem_h, sem_t, sem_c):
    wid = lax.axis_index("s") * n_cores + lax.axis_index("c")
    b_per_w = n_chunks * _CHUNK
    lane = lax.iota(jnp.int32, _LANES)

    def chunk_body(ci, _):
        base = wid * b_per_w + ci * _CHUNK
        pltpu.sync_copy(h_idx.at[pl.ds(base, _CHUNK)], hidx_v)
        pltpu.sync_copy(r_idx.at[pl.ds(base, _CHUNK)], ridx_v)
        pltpu.sync_copy(t_idx.at[pl.ds(base, _CHUNK)], tidx_v)
        ch = pltpu.async_copy(ent.at[hidx_v], h_rows, sem_h)
        ct = pltpu.async_copy(ent.at[tidx_v], t_rows, sem_t)
        cc = pltpu.async_copy(cs.at[ridx_v], cs_rows, sem_c)
        ch.wait()
        ct.wait()
        cc.wait()

        def group_body(g, _):
            for e0 in range(_LANES):
                e = g * _LANES + e0
                acc = jnp.zeros((_LANES,), jnp.float32)
                for j in range(_HALF // _LANES):
                    re_sl = pl.ds(j * _LANES, _LANES)
                    im_sl = pl.ds(_HALF + j * _LANES, _LANES)
                    h_re = h_rows[e, re_sl]
                    h_im = h_rows[e, im_sl]
                    c = cs_rows[e, re_sl]
                    s = cs_rows[e, im_sl]
                    d_re = h_re * c - h_im * s - t_rows[e, re_sl]
                    d_im = h_re * s + h_im * c - t_rows[e, im_sl]
                    sq = d_re * d_re + d_im * d_im + 1e-9
                    acc = acc + _sqrt_sc(sq)
                acc_scr[e0, :] = acc
            ovec = jnp.zeros((_LANES,), jnp.float32)
            for k in range(_LANES):
                col = jnp.full((_LANES,), k, jnp.int32)
                ovec = ovec + plsc.load_gather(acc_scr, [lane, col])
            out_v[pl.ds(g * _LANES, _LANES)] = ovec
            return 0

        lax.fori_loop(0, _CHUNK // _LANES, group_body, 0)
        pltpu.sync_copy(out_v, out.at[pl.ds(base, _CHUNK)])
        return 0

    lax.fori_loop(0, n_chunks, chunk_body, 0)


def kernel(h_idx, r_idx, t_idx, entity_emb, relation_emb):
    batch = h_idx.shape[0]
    cs = _make_trig_table(relation_emb)
    mesh = plsc.VectorSubcoreMesh(core_axis_name="c", subcore_axis_name="s")
    nw = mesh.num_cores * mesh.num_subcores
    n_chunks = batch // (nw * _CHUNK)

    run = pl.kernel(
        functools.partial(_sc_body, n_chunks, mesh.num_cores),
        out_type=jax.ShapeDtypeStruct((batch,), jnp.float32),
        mesh=mesh,
        compiler_params=pltpu.CompilerParams(needs_layout_passes=False),
        scratch_types=[
            pltpu.VMEM((_CHUNK,), jnp.int32),
            pltpu.VMEM((_CHUNK,), jnp.int32),
            pltpu.VMEM((_CHUNK,), jnp.int32),
            pltpu.VMEM((_CHUNK, _DIM), jnp.float32),
            pltpu.VMEM((_CHUNK, _DIM), jnp.float32),
            pltpu.VMEM((_CHUNK, _DIM), jnp.float32),
            pltpu.VMEM((_CHUNK,), jnp.float32),
            pltpu.VMEM((_LANES, _LANES), jnp.float32),
            pltpu.SemaphoreType.DMA,
            pltpu.SemaphoreType.DMA,
            pltpu.SemaphoreType.DMA,
        ],
    )
    return run(h_idx.astype(jnp.int32), r_idx.astype(jnp.int32),
               t_idx.astype(jnp.int32), entity_emb, cs)


# speedup vs baseline: 2.6014x; 1.1904x over previous
import functools

import jax
import jax.numpy as jnp
from jax import lax
from jax.experimental import pallas as pl
from jax.experimental.pallas import tpu as pltpu
from jax.experimental.pallas import tpu_sc as plsc

_LANES = 16
_HALF = 64
_DIM = 128
_CHUNK = 128


def _trig_body(rel_ref, out_ref):
    p = rel_ref[...]
    out_ref[...] = jnp.concatenate([jnp.cos(p), jnp.sin(p)], axis=-1)


def _make_trig_table(relation_emb):
    n, hd = relation_emb.shape
    return pl.pallas_call(
        _trig_body,
        out_shape=jax.ShapeDtypeStruct((n, 2 * hd), jnp.float32),
    )(relation_emb)


def _sqrt_sc(x):
    i = lax.bitcast_convert_type(x, jnp.int32)
    i = 0x5F375A86 - (i >> 1)
    y = lax.bitcast_convert_type(i, jnp.float32)
    y = y * (1.5 - 0.5 * x * y * y)
    return x * y


def _sc_body(n_chunks, n_cores, h_idx, r_idx, t_idx, ent, cs, out,
             hidx_v, ridx_v, tidx_v, h_rows, t_rows, cs_rows, out_v,
             acc_scr, sems):
    wid = lax.axis_index("s") * n_cores + lax.axis_index("c")
    b_per_w = n_chunks * _CHUNK
    lane = lax.iota(jnp.int32, _LANES)

    base_w = wid * b_per_w
    pltpu.sync_copy(h_idx.at[pl.ds(base_w, b_per_w)], hidx_v)
    pltpu.sync_copy(r_idx.at[pl.ds(base_w, b_per_w)], ridx_v)
    pltpu.sync_copy(t_idx.at[pl.ds(base_w, b_per_w)], tidx_v)

    def fire(ci, slot):
        sl = pl.ds(ci * _CHUNK, _CHUNK)
        dh = pltpu.async_copy(ent.at[hidx_v.at[sl]], h_rows[slot],
                              sems[3 * slot])
        dt = pltpu.async_copy(ent.at[tidx_v.at[sl]], t_rows[slot],
                              sems[3 * slot + 1])
        dc = pltpu.async_copy(cs.at[ridx_v.at[sl]], cs_rows[slot],
                              sems[3 * slot + 2])
        return (dh, dt, dc)

    def compute_chunk(slot):
        hr, tr, cr = h_rows[slot], t_rows[slot], cs_rows[slot]

        def group_body(g, _):
            for e0 in range(_LANES):
                e = g * _LANES + e0
                acc = jnp.zeros((_LANES,), jnp.float32)
                for j in range(_HALF // _LANES):
                    re_sl = pl.ds(j * _LANES, _LANES)
                    im_sl = pl.ds(_HALF + j * _LANES, _LANES)
                    h_re = hr[e, re_sl]
                    h_im = hr[e, im_sl]
                    c = cr[e, re_sl]
                    s = cr[e, im_sl]
                    d_re = h_re * c - h_im * s - tr[e, re_sl]
                    d_im = h_re * s + h_im * c - tr[e, im_sl]
                    sq = d_re * d_re + d_im * d_im + 1e-9
                    acc = acc + _sqrt_sc(sq)
                acc_scr[e0, :] = acc
            ovec = jnp.zeros((_LANES,), jnp.float32)
            for k in range(_LANES):
                col = jnp.full((_LANES,), k, jnp.int32)
                ovec = ovec + plsc.load_gather(acc_scr, [lane, col])
            out_v[pl.ds(g * _LANES, _LANES)] = ovec
            return 0

        lax.fori_loop(0, _CHUNK // _LANES, group_body, 0)

    pending = fire(0, 0)
    for ci in range(n_chunks):
        slot = ci & 1
        nxt = fire(ci + 1, 1 - slot) if ci + 1 < n_chunks else None
        for d in pending:
            d.wait()
        compute_chunk(slot)
        pltpu.sync_copy(out_v, out.at[pl.ds(base_w + ci * _CHUNK, _CHUNK)])
        pending = nxt


def kernel(h_idx, r_idx, t_idx, entity_emb, relation_emb):
    batch = h_idx.shape[0]
    cs = _make_trig_table(relation_emb)
    mesh = plsc.VectorSubcoreMesh(core_axis_name="c", subcore_axis_name="s")
    nw = mesh.num_cores * mesh.num_subcores
    n_chunks = batch // (nw * _CHUNK)

    run = pl.kernel(
        functools.partial(_sc_body, n_chunks, mesh.num_cores),
        out_type=jax.ShapeDtypeStruct((batch,), jnp.float32),
        mesh=mesh,
        compiler_params=pltpu.CompilerParams(needs_layout_passes=False),
        scratch_types=[
            pltpu.VMEM((n_chunks * _CHUNK,), jnp.int32),
            pltpu.VMEM((n_chunks * _CHUNK,), jnp.int32),
            pltpu.VMEM((n_chunks * _CHUNK,), jnp.int32),
            [pltpu.VMEM((_CHUNK, _DIM), jnp.float32) for _ in range(2)],
            [pltpu.VMEM((_CHUNK, _DIM), jnp.float32) for _ in range(2)],
            [pltpu.VMEM((_CHUNK, _DIM), jnp.float32) for _ in range(2)],
            pltpu.VMEM((_CHUNK,), jnp.float32),
            pltpu.VMEM((_LANES, _LANES), jnp.float32),
            [pltpu.SemaphoreType.DMA for _ in range(6)],
        ],
    )
    return run(h_idx.astype(jnp.int32), r_idx.astype(jnp.int32),
               t_idx.astype(jnp.int32), entity_emb, cs)


# speedup vs baseline: 3.2033x; 1.2314x over previous
import functools

import jax
import jax.numpy as jnp
from jax import lax
from jax.experimental import pallas as pl
from jax.experimental.pallas import tpu as pltpu
from jax.experimental.pallas import tpu_sc as plsc

_LANES = 16
_HALF = 64
_DIM = 128
_CHUNK = 128


def _trig_body(rel_ref, out_ref):
    p = rel_ref[...]
    out_ref[...] = jnp.concatenate([jnp.cos(p), jnp.sin(p)], axis=-1)


def _make_trig_table(relation_emb):
    n, hd = relation_emb.shape
    return pl.pallas_call(
        _trig_body,
        out_shape=jax.ShapeDtypeStruct((n, 2 * hd), jnp.float32),
    )(relation_emb)


def _sqrt_sc(x):
    i = lax.bitcast_convert_type(x, jnp.int32)
    i = 0x5F375A86 - (i >> 1)
    y = lax.bitcast_convert_type(i, jnp.float32)
    y = y * (1.5 - 0.5 * x * y * y)
    return x * y


def _sc_body(n_chunks, n_cores, h_idx, r_idx, t_idx, ent, cs, out,
             hidx_v, ridx_v, tidx_v, h_rows, t_rows, cs_rows, out_v,
             acc_scr, sems):
    wid = lax.axis_index("s") * n_cores + lax.axis_index("c")
    b_per_w = n_chunks * _CHUNK
    lane = lax.iota(jnp.int32, _LANES)

    base_w = wid * b_per_w
    pltpu.sync_copy(h_idx.at[pl.ds(base_w, b_per_w)], hidx_v)
    pltpu.sync_copy(r_idx.at[pl.ds(base_w, b_per_w)], ridx_v)
    pltpu.sync_copy(t_idx.at[pl.ds(base_w, b_per_w)], tidx_v)

    def fire(ci, slot):
        sl = pl.ds(ci * _CHUNK, _CHUNK)
        dh = pltpu.async_copy(ent.at[hidx_v.at[sl]], h_rows[slot],
                              sems[3 * slot])
        dt = pltpu.async_copy(ent.at[tidx_v.at[sl]], t_rows[slot],
                              sems[3 * slot + 1])
        dc = pltpu.async_copy(cs.at[ridx_v.at[sl]], cs_rows[slot],
                              sems[3 * slot + 2])
        return (dh, dt, dc)

    def compute_chunk(slot):
        hr, tr, cr = h_rows[slot], t_rows[slot], cs_rows[slot]
        @plsc.parallel_loop(0, _CHUNK // _LANES)
        def group_body(g):
            for e0 in range(_LANES):
                e = g * _LANES + e0
                acc = jnp.zeros((_LANES,), jnp.float32)
                for j in range(_HALF // _LANES):
                    re_sl = pl.ds(j * _LANES, _LANES)
                    im_sl = pl.ds(_HALF + j * _LANES, _LANES)
                    h_re = hr[e, re_sl]
                    h_im = hr[e, im_sl]
                    c = cr[e, re_sl]
                    s = cr[e, im_sl]
                    d_re = h_re * c - h_im * s - tr[e, re_sl]
                    d_im = h_re * s + h_im * c - tr[e, im_sl]
                    sq = d_re * d_re + d_im * d_im
                    acc = acc + _sqrt_sc(sq)
                acc_scr[e, :] = acc
            ovec = jnp.zeros((_LANES,), jnp.float32)
            row = g * _LANES + lane
            for k in range(_LANES):
                col = jnp.full((_LANES,), k, jnp.int32)
                ovec = ovec + plsc.load_gather(acc_scr, [row, col])
            out_v[pl.ds(g * _LANES, _LANES)] = ovec

    pending = fire(0, 0)
    for ci in range(n_chunks):
        slot = ci & 1
        nxt = fire(ci + 1, 1 - slot) if ci + 1 < n_chunks else None
        for d in pending:
            d.wait()
        compute_chunk(slot)
        pltpu.sync_copy(out_v, out.at[pl.ds(base_w + ci * _CHUNK, _CHUNK)])
        pending = nxt


def kernel(h_idx, r_idx, t_idx, entity_emb, relation_emb):
    batch = h_idx.shape[0]
    cs = _make_trig_table(relation_emb)
    mesh = plsc.VectorSubcoreMesh(core_axis_name="c", subcore_axis_name="s")
    nw = mesh.num_cores * mesh.num_subcores
    n_chunks = batch // (nw * _CHUNK)

    run = pl.kernel(
        functools.partial(_sc_body, n_chunks, mesh.num_cores),
        out_type=jax.ShapeDtypeStruct((batch,), jnp.float32),
        mesh=mesh,
        compiler_params=pltpu.CompilerParams(needs_layout_passes=False),
        scratch_types=[
            pltpu.VMEM((n_chunks * _CHUNK,), jnp.int32),
            pltpu.VMEM((n_chunks * _CHUNK,), jnp.int32),
            pltpu.VMEM((n_chunks * _CHUNK,), jnp.int32),
            [pltpu.VMEM((_CHUNK, _DIM), jnp.float32) for _ in range(2)],
            [pltpu.VMEM((_CHUNK, _DIM), jnp.float32) for _ in range(2)],
            [pltpu.VMEM((_CHUNK, _DIM), jnp.float32) for _ in range(2)],
            pltpu.VMEM((_CHUNK,), jnp.float32),
            pltpu.VMEM((_CHUNK, _LANES), jnp.float32),
            [pltpu.SemaphoreType.DMA for _ in range(6)],
        ],
    )
    return run(h_idx.astype(jnp.int32), r_idx.astype(jnp.int32),
               t_idx.astype(jnp.int32), entity_emb, cs)
